# baseline (speedup 1.0000x reference)
import numpy as np
import jax
import jax.numpy as jnp
from jax.experimental import pallas as pl
from jax.experimental.pallas import tpu as pltpu

_HEADS = 8
_DH = 64
_INNER = _HEADS * _DH

_DOT_T = (((1,), (1,)), ((), ()))


def _gelu_exact(v):
    return 0.5 * v * (1.0 + jax.lax.erf(v * np.float32(0.7071067811865476)))


def _attn_kernel(x_hbm, wqkv_hbm, wo_hbm, b_hbm, o_hbm,
                 x_v, wv_v, wo_v, b_v, o_v,
                 sx, swv, swo, sb, so0, so1):
    cx = pltpu.make_async_copy(x_hbm, x_v, sx)
    cwv = pltpu.make_async_copy(
        wqkv_hbm.at[pl.ds(2 * _INNER, _INNER), :], wv_v, swv)
    cwo = pltpu.make_async_copy(wo_hbm, wo_v, swo)
    cb = pltpu.make_async_copy(b_hbm, b_v, sb)
    cx.start()
    cwv.start()
    cwo.start()
    cb.start()

    cx.wait()
    cwv.wait()
    wv = wv_v[:]
    g0 = _gelu_exact(
        jax.lax.dot_general(x_v[0], wv, _DOT_T,
                            preferred_element_type=jnp.float32)
    )
    g1 = _gelu_exact(
        jax.lax.dot_general(x_v[1], wv, _DOT_T,
                            preferred_element_type=jnp.float32)
    )

    def blocks(g, heads):
        return [g[:, h * _DH:(h + 1) * _DH] for h in heads]

    u0 = jnp.concatenate(blocks(g0, (0, 2, 4, 6)) + blocks(g1, (0, 2, 4, 6)),
                         axis=1)
    u1 = jnp.concatenate(blocks(g0, (1, 3, 5, 7)) + blocks(g1, (1, 3, 5, 7)),
                         axis=1)

    cwo.wait()
    cb.wait()
    wo = wo_v[:]
    b = b_v[0, :]
    o_v[0] = jax.lax.dot_general(u0, wo, _DOT_T,
                                 preferred_element_type=jnp.float32) + b
    co0 = pltpu.make_async_copy(o_v.at[0], o_hbm.at[0], so0)
    co0.start()
    o_v[1] = jax.lax.dot_general(u1, wo, _DOT_T,
                                 preferred_element_type=jnp.float32) + b
    co1 = pltpu.make_async_copy(o_v.at[1], o_hbm.at[1], so1)
    co1.start()
    co0.wait()
    co1.wait()


def kernel(x, grad, W_qkv, W_out, b_out):
    del grad
    hbm = pl.BlockSpec(memory_space=pltpu.MemorySpace.HBM)
    out = pl.pallas_call(
        _attn_kernel,
        out_shape=jax.ShapeDtypeStruct((2, 196, 512), jnp.float32),
        in_specs=[hbm, hbm, hbm, hbm],
        out_specs=hbm,
        scratch_shapes=[
            pltpu.VMEM((2, 196, 512), jnp.float32),
            pltpu.VMEM((_INNER, 512), jnp.float32),
            pltpu.VMEM((512, _INNER), jnp.float32),
            pltpu.VMEM((1, 512), jnp.float32),
            pltpu.VMEM((2, 196, 512), jnp.float32),
            pltpu.SemaphoreType.DMA,
            pltpu.SemaphoreType.DMA,
            pltpu.SemaphoreType.DMA,
            pltpu.SemaphoreType.DMA,
            pltpu.SemaphoreType.DMA,
            pltpu.SemaphoreType.DMA,
        ],
    )(x[0], W_qkv, W_out, b_out.reshape(1, 512))
    return out[None]
